# Initial kernel scaffold; baseline (speedup 1.0000x reference)
import jax
import jax.numpy as jnp
from jax.experimental import pallas as pl


def kernel(x, edge_index, edge_attr, W_edge0, W_l0, b_l0, W_r0, W_edge1, W_l1, b_l1, W_r1):
    raise NotImplementedError("write your pallas kernel here")


# speedup vs baseline: 4.4759x; 4.4759x over previous
import functools

import jax
import jax.numpy as jnp
from jax import lax
from jax.experimental import pallas as pl
from jax.experimental.pallas import tpu as pltpu
from jax.experimental.pallas import tpu_sc as plsc

N = 10000
E = 320000
D = 128
ED = 16

NC = 2
NS = 16
NW = NC * NS
EPW = E // NW
CH = 80
NCH = EPW // CH
NP = 10240
RPS = NP // NS

_MESH = dict(core_axis_name="c", subcore_axis_name="s")


def _sc_agg_msg():
    @functools.partial(
        pl.kernel,
        out_type=jax.ShapeDtypeStruct((NC, NP, D), jnp.float32),
        mesh=plsc.VectorSubcoreMesh(**_MESH),
        scratch_types=[
            pltpu.VMEM_SHARED((NP, D), jnp.float32),
            pltpu.VMEM((CH,), jnp.int32),
            pltpu.VMEM((CH,), jnp.int32),
            pltpu.VMEM((CH, D), jnp.float32),
        ],
    )
    def agg(x_hbm, row_hbm, col_hbm, z128_hbm, p_out,
            acc_p, row_v, col_v, msg_v):
        c = lax.axis_index("c")
        s = lax.axis_index("s")
        wid = s * NC + c
        r0 = s * RPS

        pltpu.sync_copy(z128_hbm.at[pl.ds(r0, RPS)], acc_p.at[pl.ds(r0, RPS)])
        plsc.subcore_barrier()

        def body(j, carry):
            pltpu.sync_copy(row_hbm.at[wid, j], row_v)
            pltpu.sync_copy(col_hbm.at[wid, j], col_v)
            pltpu.sync_copy(x_hbm.at[row_v], msg_v)
            pltpu.sync_copy(msg_v, acc_p.at[col_v], add=True)
            return carry

        lax.fori_loop(0, NCH, body, 0)
        plsc.subcore_barrier()

        pltpu.sync_copy(acc_p.at[pl.ds(r0, RPS)], p_out.at[c, pl.ds(r0, RPS)])

    return agg


def _sc_agg_payload():
    @functools.partial(
        pl.kernel,
        out_type=jax.ShapeDtypeStruct((NC, NP, D), jnp.float32),
        mesh=plsc.VectorSubcoreMesh(**_MESH),
        scratch_types=[
            pltpu.VMEM_SHARED((NP, D), jnp.float32),
            pltpu.VMEM((CH,), jnp.int32),
            pltpu.VMEM((CH, D), jnp.float32),
        ],
    )
    def agg(pay_hbm, col_hbm, z128_hbm, p_out, acc_p, col_v, pay_v):
        c = lax.axis_index("c")
        s = lax.axis_index("s")
        wid = s * NC + c
        r0 = s * RPS

        pltpu.sync_copy(z128_hbm.at[pl.ds(r0, RPS)], acc_p.at[pl.ds(r0, RPS)])
        plsc.subcore_barrier()

        def body(j, carry):
            pltpu.sync_copy(col_hbm.at[wid, j], col_v)
            pltpu.sync_copy(pay_hbm.at[wid, j], pay_v)
            pltpu.sync_copy(pay_v, acc_p.at[col_v], add=True)
            return carry

        lax.fori_loop(0, NCH, body, 0)
        plsc.subcore_barrier()

        pltpu.sync_copy(acc_p.at[pl.ds(r0, RPS)], p_out.at[c, pl.ds(r0, RPS)])

    return agg


_BN = 1000


def _tc_dense_body(p_ref, pay_ref, h_ref, we_ref, wl_ref, wr_ref,
                   b_ref, o_ref, *, relu):
    p = p_ref[0] + p_ref[1]
    pay = pay_ref[0] + pay_ref[1]
    ea = pay[:, :ED]
    deg = pay[:, ED:ED + 1]
    inv = 1.0 / jnp.maximum(deg, 1.0)
    dn = (((1,), (1,)), ((), ()))
    e_term = lax.dot_general(ea, we_ref[...], dn,
                             preferred_element_type=jnp.float32)
    su = (p + e_term) * inv
    out = (lax.dot_general(su, wl_ref[...], dn,
                           preferred_element_type=jnp.float32)
           + lax.dot_general(h_ref[...], wr_ref[...], dn,
                             preferred_element_type=jnp.float32)
           + b_ref[...])
    if relu:
        out = jnp.maximum(out, 0.0)
    o_ref[...] = out


def _tc_dense(relu):
    return pl.pallas_call(
        functools.partial(_tc_dense_body, relu=relu),
        grid=(N // _BN,),
        in_specs=[
            pl.BlockSpec((NC, _BN, D), lambda i: (0, i, 0)),
            pl.BlockSpec((NC, _BN, D), lambda i: (0, i, 0)),
            pl.BlockSpec((_BN, D), lambda i: (i, 0)),
            pl.BlockSpec((D, ED), lambda i: (0, 0)),
            pl.BlockSpec((D, D), lambda i: (0, 0)),
            pl.BlockSpec((D, D), lambda i: (0, 0)),
            pl.BlockSpec((1, D), lambda i: (0, 0)),
        ],
        out_specs=pl.BlockSpec((_BN, D), lambda i: (i, 0)),
        out_shape=jax.ShapeDtypeStruct((N, D), jnp.float32),
    )


def kernel(x, edge_index, edge_attr, W_edge0, W_l0, b_l0, W_r0,
           W_edge1, W_l1, b_l1, W_r1):
    row3 = edge_index[0].reshape(NW, NCH, CH)
    col3 = edge_index[1].reshape(NW, NCH, CH)
    pay = jnp.concatenate(
        [edge_attr, jnp.ones((E, 1), jnp.float32),
         jnp.zeros((E, D - ED - 1), jnp.float32)], axis=1)
    pay4 = pay.reshape(NW, NCH, CH, D)
    z128 = jnp.zeros((NP, D), jnp.float32)

    pp = _sc_agg_payload()(pay4, col3, z128)
    p0 = _sc_agg_msg()(x, row3, col3, z128)
    h1 = _tc_dense(True)(p0, pp, x, W_edge0, W_l0, W_r0, b_l0.reshape(1, D))
    p1 = _sc_agg_msg()(h1, row3, col3, z128)
    out = _tc_dense(False)(p1, pp, h1, W_edge1, W_l1, W_r1,
                           b_l1.reshape(1, D))
    return out


# speedup vs baseline: 8.9575x; 2.0013x over previous
import functools

import jax
import jax.numpy as jnp
from jax import lax
from jax.experimental import pallas as pl
from jax.experimental.pallas import tpu as pltpu
from jax.experimental.pallas import tpu_sc as plsc

N = 10000
E = 320000
D = 128
ED = 16

NC = 2
NS = 16
NW = NC * NS
EPW = E // NW
CH = 80
NCH = EPW // CH
NB = 4
NOUT = NCH // NB
NREM = NCH - NOUT * NB
NP = 10240
RPS = NP // NS

_MESH = dict(core_axis_name="c", subcore_axis_name="s")


def _msg_scratch():
    return (
        [pltpu.VMEM_SHARED((NP, D), jnp.float32)]
        + [pltpu.VMEM((CH,), jnp.int32) for _ in range(2 * NB)]
        + [pltpu.VMEM((CH, D), jnp.float32) for _ in range(NB)]
        + [pltpu.SemaphoreType.DMA for _ in range(3)]
    )


def _sc_agg_msg():
    @functools.partial(
        pl.kernel,
        out_type=jax.ShapeDtypeStruct((NC, NP, D), jnp.float32),
        mesh=plsc.VectorSubcoreMesh(**_MESH),
        scratch_types=_msg_scratch(),
    )
    def agg(x_hbm, row_hbm, col_hbm, z128_hbm, p_out, acc_p, *bufs):
        row_v = bufs[0:NB]
        col_v = bufs[NB:2 * NB]
        msg_v = bufs[2 * NB:3 * NB]
        sem_i, sem_g, sem_s = bufs[3 * NB:]
        c = lax.axis_index("c")
        s = lax.axis_index("s")
        wid = s * NC + c
        r0 = s * RPS

        pltpu.sync_copy(z128_hbm.at[pl.ds(r0, RPS)], acc_p.at[pl.ds(r0, RPS)])
        plsc.subcore_barrier()

        e_base = wid * EPW

        def do_batch(j0, nb):
            ic = [None] * nb
            for b in range(nb):
                e0 = e_base + (j0 + b) * CH
                ic[b] = (
                    pltpu.async_copy(row_hbm.at[pl.ds(e0, CH)], row_v[b], sem_i),
                    pltpu.async_copy(col_hbm.at[pl.ds(e0, CH)], col_v[b], sem_i),
                )
            gc = [None] * nb
            for b in range(nb):
                ic[b][0].wait()
                ic[b][1].wait()
                gc[b] = pltpu.async_copy(x_hbm.at[row_v[b]], msg_v[b], sem_g)
            sc = [None] * nb
            for b in range(nb):
                gc[b].wait()
                sc[b] = pltpu.async_copy(msg_v[b], acc_p.at[col_v[b]], sem_s,
                                         add=True)
            for b in range(nb):
                sc[b].wait()

        def body(i, carry):
            do_batch(i * NB, NB)
            return carry

        lax.fori_loop(0, NOUT, body, 0)
        do_batch(NOUT * NB, NREM)
        plsc.subcore_barrier()

        pltpu.sync_copy(acc_p.at[pl.ds(r0, RPS)], p_out.at[c, pl.ds(r0, RPS)])

    return agg


def _sc_agg_payload():
    @functools.partial(
        pl.kernel,
        out_type=jax.ShapeDtypeStruct((NC, NP, D), jnp.float32),
        mesh=plsc.VectorSubcoreMesh(**_MESH),
        scratch_types=(
            [pltpu.VMEM_SHARED((NP, D), jnp.float32)]
            + [pltpu.VMEM((CH,), jnp.int32) for _ in range(NB)]
            + [pltpu.VMEM((CH, D), jnp.float32) for _ in range(NB)]
            + [pltpu.SemaphoreType.DMA for _ in range(3)]
        ),
    )
    def agg(pay_hbm, col_hbm, z128_hbm, p_out, acc_p, *bufs):
        col_v = bufs[0:NB]
        pay_v = bufs[NB:2 * NB]
        sem_i, sem_g, sem_s = bufs[2 * NB:]
        c = lax.axis_index("c")
        s = lax.axis_index("s")
        wid = s * NC + c
        r0 = s * RPS

        pltpu.sync_copy(z128_hbm.at[pl.ds(r0, RPS)], acc_p.at[pl.ds(r0, RPS)])
        plsc.subcore_barrier()

        e_base = wid * EPW

        def do_batch(j0, nb):
            ic = [None] * nb
            gc = [None] * nb
            for b in range(nb):
                e0 = e_base + (j0 + b) * CH
                ic[b] = pltpu.async_copy(col_hbm.at[pl.ds(e0, CH)], col_v[b],
                                         sem_i)
                gc[b] = pltpu.async_copy(pay_hbm.at[pl.ds(e0, CH)], pay_v[b],
                                         sem_g)
            sc = [None] * nb
            for b in range(nb):
                ic[b].wait()
                gc[b].wait()
                sc[b] = pltpu.async_copy(pay_v[b], acc_p.at[col_v[b]], sem_s,
                                         add=True)
            for b in range(nb):
                sc[b].wait()

        def body(i, carry):
            do_batch(i * NB, NB)
            return carry

        lax.fori_loop(0, NOUT, body, 0)
        do_batch(NOUT * NB, NREM)
        plsc.subcore_barrier()

        pltpu.sync_copy(acc_p.at[pl.ds(r0, RPS)], p_out.at[c, pl.ds(r0, RPS)])

    return agg


_BN = 1000


def _tc_dense_body(p_ref, pay_ref, h_ref, we_ref, wl_ref, wr_ref,
                   b_ref, o_ref, *, relu):
    p = p_ref[0] + p_ref[1]
    pay = pay_ref[0] + pay_ref[1]
    ea = pay[:, :ED]
    deg = pay[:, ED:ED + 1]
    inv = 1.0 / jnp.maximum(deg, 1.0)
    dn = (((1,), (1,)), ((), ()))
    e_term = lax.dot_general(ea, we_ref[...], dn,
                             preferred_element_type=jnp.float32)
    su = (p + e_term) * inv
    out = (lax.dot_general(su, wl_ref[...], dn,
                           preferred_element_type=jnp.float32)
           + lax.dot_general(h_ref[...], wr_ref[...], dn,
                             preferred_element_type=jnp.float32)
           + b_ref[...])
    if relu:
        out = jnp.maximum(out, 0.0)
    o_ref[...] = out


def _tc_dense(relu):
    return pl.pallas_call(
        functools.partial(_tc_dense_body, relu=relu),
        grid=(N // _BN,),
        in_specs=[
            pl.BlockSpec((NC, _BN, D), lambda i: (0, i, 0)),
            pl.BlockSpec((NC, _BN, D), lambda i: (0, i, 0)),
            pl.BlockSpec((_BN, D), lambda i: (i, 0)),
            pl.BlockSpec((D, ED), lambda i: (0, 0)),
            pl.BlockSpec((D, D), lambda i: (0, 0)),
            pl.BlockSpec((D, D), lambda i: (0, 0)),
            pl.BlockSpec((1, D), lambda i: (0, 0)),
        ],
        out_specs=pl.BlockSpec((_BN, D), lambda i: (i, 0)),
        out_shape=jax.ShapeDtypeStruct((N, D), jnp.float32),
    )


def kernel(x, edge_index, edge_attr, W_edge0, W_l0, b_l0, W_r0,
           W_edge1, W_l1, b_l1, W_r1):
    row1 = edge_index[0]
    col1 = edge_index[1]
    pay = jnp.concatenate(
        [edge_attr, jnp.ones((E, 1), jnp.float32),
         jnp.zeros((E, D - ED - 1), jnp.float32)], axis=1)
    z128 = jnp.zeros((NP, D), jnp.float32)

    pp = _sc_agg_payload()(pay, col1, z128)
    p0 = _sc_agg_msg()(x, row1, col1, z128)
    h1 = _tc_dense(True)(p0, pp, x, W_edge0, W_l0, W_r0, b_l0.reshape(1, D))
    p1 = _sc_agg_msg()(h1, row1, col1, z128)
    out = _tc_dense(False)(p1, pp, h1, W_edge1, W_l1, W_r1,
                           b_l1.reshape(1, D))
    return out
